# Initial kernel scaffold; baseline (speedup 1.0000x reference)
import jax
import jax.numpy as jnp
from jax.experimental import pallas as pl


def kernel(x, edge_index, W1_l, b1, W1_r, W2_l, b2, W2_r):
    raise NotImplementedError("write your pallas kernel here")


# speedup vs baseline: 5.6463x; 5.6463x over previous
import functools

import jax
import jax.numpy as jnp
from jax import lax
from jax.experimental import pallas as pl
from jax.experimental.pallas import tpu as pltpu
from jax.experimental.pallas import tpu_sc as plsc

N_NODES = 10000
N_PAD = 10240
D = 128
E = 320000
NC, NS = 2, 16
NW = NC * NS
EDGES_PER_TILE = E // NW
CHUNK = 80
NCHUNK = EDGES_PER_TILE // CHUNK
ROWS_PER_TILE = N_PAD // NS


def _zero_rows(rows_v):
    zeros16 = jnp.zeros((16,), jnp.float32)

    def body(r, _):
        for c in range(D // 16):
            rows_v[r, pl.ds(c * 16, 16)] = zeros16
        return 0

    lax.fori_loop(0, CHUNK, body, 0)


def _sc_agg_body(with_deg, x_hbm, src_hbm, dst_hbm, *rest):
    if with_deg:
        out_agg, out_deg, src_v, dst_v, rows_v, deg_v, agg_sh, sem = rest
    else:
        out_agg, src_v, dst_v, rows_v, deg_v, agg_sh, sem = rest
    cid = lax.axis_index("c")
    sid = lax.axis_index("s")
    g = cid * NS + sid

    _zero_rows(rows_v)
    for b in range(ROWS_PER_TILE // CHUNK):
        pltpu.sync_copy(rows_v,
                        agg_sh.at[pl.ds(sid * ROWS_PER_TILE + b * CHUNK, CHUNK)])
    if with_deg:
        zeros16 = jnp.zeros((16,), jnp.float32)

        def zbody(k, _):
            deg_v[pl.ds(k * 16, 16)] = zeros16
            return 0

        lax.fori_loop(0, N_PAD // 16, zbody, 0)
    plsc.subcore_barrier()

    ones16 = jnp.ones((16,), jnp.float32)
    ebase = g * EDGES_PER_TILE

    def body(j, _):
        base = ebase + j * CHUNK
        pltpu.sync_copy(src_hbm.at[pl.ds(base, CHUNK)], src_v)
        pltpu.sync_copy(dst_hbm.at[pl.ds(base, CHUNK)], dst_v)
        pltpu.async_copy(x_hbm.at[src_v], rows_v, sem).wait()
        pltpu.sync_copy(rows_v, agg_sh.at[dst_v], add=True)
        if with_deg:
            for k in range(CHUNK // 16):
                d16 = dst_v[pl.ds(k * 16, 16)]
                plsc.addupdate_scatter(deg_v, [d16], ones16)
        return 0

    lax.fori_loop(0, NCHUNK, body, 0)
    plsc.subcore_barrier()

    pltpu.sync_copy(agg_sh.at[pl.ds(sid * ROWS_PER_TILE, ROWS_PER_TILE)],
                    out_agg.at[cid, pl.ds(sid * ROWS_PER_TILE, ROWS_PER_TILE)])
    if with_deg:
        pltpu.sync_copy(deg_v, out_deg.at[g])


def _make_sc_agg(with_deg):
    out_type = [jax.ShapeDtypeStruct((NC, N_PAD, D), jnp.float32)]
    if with_deg:
        out_type.append(jax.ShapeDtypeStruct((NW, N_PAD), jnp.float32))
    return pl.kernel(
        functools.partial(_sc_agg_body, with_deg),
        out_type=tuple(out_type),
        mesh=plsc.VectorSubcoreMesh(core_axis_name="c", subcore_axis_name="s"),
        scratch_types=[
            pltpu.VMEM((CHUNK,), jnp.int32),
            pltpu.VMEM((CHUNK,), jnp.int32),
            pltpu.VMEM((CHUNK, D), jnp.float32),
            pltpu.VMEM((N_PAD,), jnp.float32),
            pltpu.VMEM_SHARED((N_PAD, D), jnp.float32),
            pltpu.SemaphoreType.DMA,
        ],
        compiler_params=pltpu.CompilerParams(needs_layout_passes=False),
        name="sc_agg_deg" if with_deg else "sc_agg",
    )


_sc_agg_deg = _make_sc_agg(True)
_sc_agg = _make_sc_agg(False)


def _tc_layer_body(relu, aggp_ref, degp_ref, x_ref, wl_ref, wr_ref, b_ref,
                   o_ref):
    deg = jnp.sum(degp_ref[...], axis=0)
    rdeg = 1.0 / jnp.maximum(deg, 1.0)
    agg = (aggp_ref[0] + aggp_ref[1]) * rdeg[:, None]
    y = jnp.dot(agg, wl_ref[...], preferred_element_type=jnp.float32)
    y = y + jnp.dot(x_ref[...], wr_ref[...], preferred_element_type=jnp.float32)
    y = y + b_ref[...]
    if relu:
        y = jnp.maximum(y, 0.0)
    o_ref[...] = y


_TC_R = 1024


def _tc_layer(aggp, degp, x, w_l, b, w_r, relu):
    return pl.pallas_call(
        functools.partial(_tc_layer_body, relu),
        grid=(N_PAD // _TC_R,),
        in_specs=[
            pl.BlockSpec((NC, _TC_R, D), lambda i: (0, i, 0)),
            pl.BlockSpec((NW, _TC_R), lambda i: (0, i)),
            pl.BlockSpec((_TC_R, D), lambda i: (i, 0)),
            pl.BlockSpec((D, D), lambda i: (0, 0)),
            pl.BlockSpec((D, D), lambda i: (0, 0)),
            pl.BlockSpec((1, D), lambda i: (0, 0)),
        ],
        out_specs=pl.BlockSpec((_TC_R, D), lambda i: (i, 0)),
        out_shape=jax.ShapeDtypeStruct((N_PAD, D), jnp.float32),
    )(aggp, degp, x, w_l, w_r, b)


def kernel(x, edge_index, W1_l, b1, W1_r, W2_l, b2, W2_r):
    src = edge_index[0].astype(jnp.int32)
    dst = edge_index[1].astype(jnp.int32)
    xp = jnp.pad(x, ((0, N_PAD - N_NODES), (0, 0)))
    agg1, degp = _sc_agg_deg(xp, src, dst)
    h = _tc_layer(agg1, degp, xp, W1_l, b1.reshape(1, D), W1_r, relu=True)
    agg2, = _sc_agg(h, src, dst)
    out = _tc_layer(agg2, degp, h, W2_l, b2.reshape(1, D), W2_r, relu=False)
    return out[:N_NODES]
